# baseline (speedup 1.0000x reference)
import jax
import jax.numpy as jnp
from jax import lax
from jax.experimental import pallas as pl
from jax.experimental.pallas import tpu as pltpu
from jax.experimental.pallas import tpu_sc as plsc

B = 16384
D = 32
NC = 2
NS = 16
NW = NC * NS
BPW = B // NW
PASS = 128
GROUPS = PASS // 16
ROWBATCH = 8


def _body(u_id_hbm, i_id_hbm, user_emb_hbm, user_bias_hbm, item_emb_hbm,
          item_bias_hbm, mean_hbm, out_hbm,
          uid_v, iid_v, urow_v, irow_v, ub_v, ib_v, out_v, mean_v, sem):
  wid = lax.axis_index("s") * NC + lax.axis_index("c")
  base = wid * BPW

  pltpu.sync_copy(u_id_hbm.at[pl.ds(base, BPW)], uid_v.at[pl.ds(0, BPW)])
  pltpu.sync_copy(i_id_hbm.at[pl.ds(base, BPW)], iid_v.at[pl.ds(0, BPW)])
  pltpu.sync_copy(mean_hbm, mean_v)

  lanes = lax.iota(jnp.int32, 16)
  zeros = jnp.zeros((16,), jnp.int32)
  mean16 = mean_v[...]

  def pass_body(h, carry):
    def fetch_batch(t, _):
      r0 = h * PASS + t * ROWBATCH
      uvec = uid_v[pl.ds(r0, 16)]
      ivec = iid_v[pl.ds(r0, 16)]
      copies = []
      for j in range(ROWBATCH):
        r = t * ROWBATCH + j
        uid = uvec[j]
        iid = ivec[j]
        copies.append(pltpu.make_async_copy(
            user_emb_hbm.at[pl.ds(uid, 1), :], urow_v.at[pl.ds(r, 1), :],
            sem))
        copies.append(pltpu.make_async_copy(
            item_emb_hbm.at[pl.ds(iid, 1), :], irow_v.at[pl.ds(r, 1), :],
            sem))
        copies.append(pltpu.make_async_copy(
            user_bias_hbm.at[pl.ds(uid, 1), :], ub_v.at[pl.ds(r, 1), :],
            sem))
        copies.append(pltpu.make_async_copy(
            item_bias_hbm.at[pl.ds(iid, 1), :], ib_v.at[pl.ds(r, 1), :],
            sem))
      for cp in copies:
        cp.start()
      for cp in copies:
        cp.wait()
      return _

    lax.fori_loop(0, PASS // ROWBATCH, fetch_batch, 0)

    def group_body(g, c2):
      rows = g * 16 + lanes
      acc = mean16
      acc = acc + plsc.load_gather(ub_v, [rows, zeros])
      acc = acc + plsc.load_gather(ib_v, [rows, zeros])
      for d in range(D):
        col = jnp.full((16,), d, jnp.int32)
        u = plsc.load_gather(urow_v, [rows, col])
        i = plsc.load_gather(irow_v, [rows, col])
        acc = acc + u * i
      out_v[pl.ds(h * PASS + g * 16, 16)] = acc
      return c2

    lax.fori_loop(0, GROUPS, group_body, 0)
    return carry

  lax.fori_loop(0, BPW // PASS, pass_body, 0)

  pltpu.sync_copy(out_v, out_hbm.at[pl.ds(base, BPW)])


@jax.jit
def kernel(u_id, i_id, user_emb, user_bias, item_emb, item_bias, mean):
  mesh = plsc.VectorSubcoreMesh(core_axis_name="c", subcore_axis_name="s")
  f = pl.kernel(
      _body,
      out_type=jax.ShapeDtypeStruct((B,), jnp.float32),
      mesh=mesh,
      scratch_types=[
          pltpu.VMEM((BPW + 8,), jnp.int32),
          pltpu.VMEM((BPW + 8,), jnp.int32),
          pltpu.VMEM((PASS, D), jnp.float32),
          pltpu.VMEM((PASS, D), jnp.float32),
          pltpu.VMEM((PASS, 1), jnp.float32),
          pltpu.VMEM((PASS, 1), jnp.float32),
          pltpu.VMEM((BPW,), jnp.float32),
          pltpu.VMEM((16,), jnp.float32),
          pltpu.SemaphoreType.DMA,
      ],
      compiler_params=pltpu.CompilerParams(needs_layout_passes=False),
  )
  mean16 = jnp.broadcast_to(mean.astype(jnp.float32), (16,))
  return f(u_id, i_id, user_emb, user_bias, item_emb, item_bias, mean16)


# speedup vs baseline: 1.0177x; 1.0177x over previous
import jax
import jax.numpy as jnp
from jax import lax
from jax.experimental import pallas as pl
from jax.experimental.pallas import tpu as pltpu
from jax.experimental.pallas import tpu_sc as plsc

B = 16384
D = 32
NC = 2
NS = 16
NW = NC * NS
BPW = B // NW
STEP = 16
NSTEP = BPW // STEP


def _fire(uvec, ivec, user_emb_hbm, user_bias_hbm, item_emb_hbm,
          item_bias_hbm, urow, irow, ub, ib, sem):
  copies = []
  for j in range(STEP):
    uid = uvec[j]
    iid = ivec[j]
    copies.append(pltpu.make_async_copy(
        user_emb_hbm.at[pl.ds(uid, 1), :], urow.at[pl.ds(j, 1), :], sem))
    copies.append(pltpu.make_async_copy(
        item_emb_hbm.at[pl.ds(iid, 1), :], irow.at[pl.ds(j, 1), :], sem))
    copies.append(pltpu.make_async_copy(
        user_bias_hbm.at[pl.ds(uid, 1), :], ub.at[pl.ds(j, 1), :], sem))
    copies.append(pltpu.make_async_copy(
        item_bias_hbm.at[pl.ds(iid, 1), :], ib.at[pl.ds(j, 1), :], sem))
  for cp in copies:
    cp.start()
  return copies


def _drain(copies):
  for cp in copies:
    cp.wait()


def _body(u_id_hbm, i_id_hbm, user_emb_hbm, user_bias_hbm, item_emb_hbm,
          item_bias_hbm, mean_hbm, out_hbm,
          uid_v, iid_v, urow0, irow0, ub0, ib0, urow1, irow1, ub1, ib1,
          out_v, mean_v, sem0, sem1):
  wid = lax.axis_index("s") * NC + lax.axis_index("c")
  base = wid * BPW

  pltpu.sync_copy(u_id_hbm.at[pl.ds(base, BPW)], uid_v)
  pltpu.sync_copy(i_id_hbm.at[pl.ds(base, BPW)], iid_v)
  pltpu.sync_copy(mean_hbm, mean_v)

  lanes = lax.iota(jnp.int32, 16)
  zeros = jnp.zeros((16,), jnp.int32)
  mean16 = mean_v[...]
  tables = (user_emb_hbm, user_bias_hbm, item_emb_hbm, item_bias_hbm)
  bufs = ((urow0, irow0, ub0, ib0, sem0), (urow1, irow1, ub1, ib1, sem1))

  def compute(urow, irow, ub, ib, t):
    acc = mean16
    acc = acc + plsc.load_gather(ub, [lanes, zeros])
    acc = acc + plsc.load_gather(ib, [lanes, zeros])
    for d in range(D):
      col = jnp.full((16,), d, jnp.int32)
      u = plsc.load_gather(urow, [lanes, col])
      i = plsc.load_gather(irow, [lanes, col])
      acc = acc + u * i
    out_v[pl.ds(t * STEP, 16)] = acc

  def step2(t, _):
    for p, (urow, irow, ub, ib, sem) in enumerate(bufs):
      tt = t * 2 + p
      uvec = uid_v[pl.ds(tt * STEP, 16)]
      ivec = iid_v[pl.ds(tt * STEP, 16)]
      cps = _fire(uvec, ivec, *tables, urow, irow, ub, ib, sem)
      _drain(cps)
      compute(urow, irow, ub, ib, tt)
    return _

  lax.fori_loop(0, NSTEP // 2, step2, 0)

  pltpu.sync_copy(out_v, out_hbm.at[pl.ds(base, BPW)])


@jax.jit
def kernel(u_id, i_id, user_emb, user_bias, item_emb, item_bias, mean):
  mesh = plsc.VectorSubcoreMesh(core_axis_name="c", subcore_axis_name="s")
  f = pl.kernel(
      _body,
      out_type=jax.ShapeDtypeStruct((B,), jnp.float32),
      mesh=mesh,
      scratch_types=[
          pltpu.VMEM((BPW,), jnp.int32),
          pltpu.VMEM((BPW,), jnp.int32),
          pltpu.VMEM((STEP, D), jnp.float32),
          pltpu.VMEM((STEP, D), jnp.float32),
          pltpu.VMEM((STEP, 1), jnp.float32),
          pltpu.VMEM((STEP, 1), jnp.float32),
          pltpu.VMEM((STEP, D), jnp.float32),
          pltpu.VMEM((STEP, D), jnp.float32),
          pltpu.VMEM((STEP, 1), jnp.float32),
          pltpu.VMEM((STEP, 1), jnp.float32),
          pltpu.VMEM((BPW,), jnp.float32),
          pltpu.VMEM((16,), jnp.float32),
          pltpu.SemaphoreType.DMA,
          pltpu.SemaphoreType.DMA,
      ],
      compiler_params=pltpu.CompilerParams(needs_layout_passes=False),
  )
  mean16 = jnp.broadcast_to(mean.astype(jnp.float32), (16,))
  return f(u_id, i_id, user_emb, user_bias, item_emb, item_bias, mean16)


# speedup vs baseline: 1.1923x; 1.1716x over previous
import functools

import jax
import jax.numpy as jnp
from jax import lax
from jax.experimental import pallas as pl
from jax.experimental.pallas import tpu as pltpu
from jax.experimental.pallas import tpu_sc as plsc

B = 16384
D = 32
NC = 2
NS = 16
NW = NC * NS
BPW = B // NW
CHUNK = 128
NCHUNK = BPW // CHUNK
GROUPS = BPW // 16


def _body(u_id_hbm, i_id_hbm, user_emb_hbm, user_bias_hbm, item_emb_hbm,
          item_bias_hbm, mean_hbm, out_hbm,
          uid_v, iid_v, urow_v, irow_v, ub_v, ib_v, out_v, mean_v, sem):
  wid = lax.axis_index("s") * NC + lax.axis_index("c")
  base = wid * BPW

  pltpu.sync_copy(u_id_hbm.at[pl.ds(base, BPW)], uid_v)
  pltpu.sync_copy(i_id_hbm.at[pl.ds(base, BPW)], iid_v)
  pltpu.sync_copy(mean_hbm, mean_v)

  copies = []
  for c in range(NCHUNK):
    s = pl.ds(c * CHUNK, CHUNK)
    copies.append(pltpu.make_async_copy(
        user_emb_hbm.at[uid_v.at[s]], urow_v.at[s, :], sem))
    copies.append(pltpu.make_async_copy(
        item_emb_hbm.at[iid_v.at[s]], irow_v.at[s, :], sem))
    copies.append(pltpu.make_async_copy(
        user_bias_hbm.at[uid_v.at[s]], ub_v.at[s], sem))
    copies.append(pltpu.make_async_copy(
        item_bias_hbm.at[iid_v.at[s]], ib_v.at[s], sem))
  for cp in copies:
    cp.start()
  for cp in copies:
    cp.wait()

  lanes = lax.iota(jnp.int32, 16)
  zeros = jnp.zeros((16,), jnp.int32)
  mean16 = mean_v[...]

  def group_body(g, carry):
    rows = g * 16 + lanes
    acc = mean16
    acc = acc + ub_v[pl.ds(g * 16, 16)]
    acc = acc + ib_v[pl.ds(g * 16, 16)]
    for d in range(D):
      col = jnp.full((16,), d, jnp.int32)
      u = plsc.load_gather(urow_v, [rows, col])
      i = plsc.load_gather(irow_v, [rows, col])
      acc = acc + u * i
    out_v[pl.ds(g * 16, 16)] = acc
    return carry

  lax.fori_loop(0, GROUPS, group_body, 0)

  pltpu.sync_copy(out_v, out_hbm.at[pl.ds(base, BPW)])


@jax.jit
def kernel(u_id, i_id, user_emb, user_bias, item_emb, item_bias, mean):
  mesh = plsc.VectorSubcoreMesh(core_axis_name="c", subcore_axis_name="s")
  f = pl.kernel(
      _body,
      out_type=jax.ShapeDtypeStruct((B,), jnp.float32),
      mesh=mesh,
      scratch_types=[
          pltpu.VMEM((BPW,), jnp.int32),
          pltpu.VMEM((BPW,), jnp.int32),
          pltpu.VMEM((BPW, D), jnp.float32),
          pltpu.VMEM((BPW, D), jnp.float32),
          pltpu.VMEM((BPW,), jnp.float32),
          pltpu.VMEM((BPW,), jnp.float32),
          pltpu.VMEM((BPW,), jnp.float32),
          pltpu.VMEM((16,), jnp.float32),
          pltpu.SemaphoreType.DMA,
      ],
      compiler_params=pltpu.CompilerParams(
          needs_layout_passes=False, use_tc_tiling_on_sc=False),
  )
  mean16 = jnp.broadcast_to(mean.astype(jnp.float32), (16,))
  return f(u_id, i_id, user_emb, user_bias.reshape(-1), item_emb,
           item_bias.reshape(-1), mean16)
